# baseline (speedup 1.0000x reference)
import functools

import jax
import jax.numpy as jnp
from jax import lax
from jax.experimental import pallas as pl
from jax.experimental.pallas import tpu as pltpu
from jax.experimental.pallas import tpu_sc as plsc

N = 10000
E = 320000
D_IN = 128
H = 16
D_OUT = 300

NC = 2
NS = 16
NW = NC * NS
E_TILE = E // NW
CHUNK = 128
CHUNKS = E_TILE // CHUNK
TAIL = E_TILE - CHUNKS * CHUNK
N_PAD = 10240
RPT = N_PAD // NS
D_OUT_PAD = 304
BM = 1024
BT = 2000

_mesh = plsc.VectorSubcoreMesh(
    core_axis_name="c", subcore_axis_name="s", num_cores=NC, num_subcores=NS)


def _edge_pass(table_h, src_v, dst_v, buf0, buf1, acc, sem0, sem1):
    def gather(j, buf, sem):
        return pltpu.async_copy(
            table_h.at[src_v.at[pl.ds(j * CHUNK, CHUNK)]], buf, sem)

    def gwait(buf, sem):
        pltpu.make_async_copy(table_h.at[pl.ds(0, CHUNK)], buf, sem).wait()

    def scat(j, buf):
        pltpu.sync_copy(buf, acc.at[dst_v.at[pl.ds(j * CHUNK, CHUNK)]],
                        add=True)

    gather(0, buf0, sem0)

    def pair(i, carry):
        j = 2 * i
        gather(j + 1, buf1, sem1)
        gwait(buf0, sem0)
        scat(j, buf0)

        @pl.when(i < CHUNKS // 2 - 1)
        def _():
            gather(j + 2, buf0, sem0)

        gwait(buf1, sem1)
        scat(j + 1, buf1)
        return carry

    lax.fori_loop(0, CHUNKS // 2, pair, 0)
    tb = CHUNKS * CHUNK
    pltpu.async_copy(
        table_h.at[src_v.at[pl.ds(tb, TAIL)]], buf0.at[pl.ds(0, TAIL)],
        sem0).wait()
    pltpu.sync_copy(buf0.at[pl.ds(0, TAIL)],
                    acc.at[dst_v.at[pl.ds(tb, TAIL)]], add=True)


def _sc_agg(table, ei, zeros):
    @functools.partial(
        pl.kernel,
        mesh=_mesh,
        out_type=jax.ShapeDtypeStruct((NC, N_PAD, H), jnp.float32),
        compiler_params=pltpu.CompilerParams(use_tc_tiling_on_sc=False),
        scratch_types=[
            pltpu.VMEM((E_TILE,), jnp.int32),
            pltpu.VMEM((E_TILE,), jnp.int32),
            pltpu.VMEM((CHUNK, H), jnp.float32),
            pltpu.VMEM((CHUNK, H), jnp.float32),
            pltpu.VMEM_SHARED((N_PAD, H), jnp.float32),
            pltpu.SemaphoreType.DMA,
            pltpu.SemaphoreType.DMA,
        ],
    )
    def body(table_h, ei_h, zeros_h, out_h, src_v, dst_v, buf0, buf1, acc,
             sem0, sem1):
        cid = lax.axis_index("c")
        sid = lax.axis_index("s")
        wid = cid * NS + sid
        row0 = sid * RPT
        e0 = wid * E_TILE
        pltpu.sync_copy(zeros_h.at[pl.ds(row0, RPT)], acc.at[pl.ds(row0, RPT)])
        pltpu.sync_copy(ei_h.at[0, pl.ds(e0, E_TILE)], src_v)
        pltpu.sync_copy(ei_h.at[1, pl.ds(e0, E_TILE)], dst_v)
        plsc.subcore_barrier()
        _edge_pass(table_h, src_v, dst_v, buf0, buf1, acc, sem0, sem1)
        plsc.subcore_barrier()
        pltpu.sync_copy(acc.at[pl.ds(row0, RPT)], out_h.at[cid, pl.ds(row0, RPT)])

    return body(table, ei, zeros)


def _nrsqrt(v):
    i = plsc.bitcast(v, jnp.int32)
    y = plsc.bitcast(jnp.int32(0x5F3759DF) - (i >> 1), jnp.float32)
    for _ in range(3):
        y = y * (1.5 - 0.5 * v * y * y)
    return y


def _sc_bn_agg(q0, q1, s, dv, params, ei, zeros):
    @functools.partial(
        pl.kernel,
        mesh=_mesh,
        out_type=[jax.ShapeDtypeStruct((NC, N_PAD, H), jnp.float32),
                  jax.ShapeDtypeStruct((N_PAD, H), jnp.float32),
                  jax.ShapeDtypeStruct((N_PAD, H), jnp.float32)],
        compiler_params=pltpu.CompilerParams(use_tc_tiling_on_sc=False,
                                             needs_layout_passes=False),
        scratch_types=[
            pltpu.VMEM((E_TILE,), jnp.int32),
            pltpu.VMEM((E_TILE,), jnp.int32),
            pltpu.VMEM((CHUNK, H), jnp.float32),
            pltpu.VMEM((CHUNK, H), jnp.float32),
            pltpu.VMEM((RPT, H), jnp.float32),
            pltpu.VMEM((RPT, H), jnp.float32),
            pltpu.VMEM((RPT, H), jnp.float32),
            pltpu.VMEM((RPT, H), jnp.float32),
            pltpu.VMEM((RPT, H), jnp.float32),
            pltpu.VMEM((8, H), jnp.float32),
            pltpu.VMEM_SHARED((N_PAD, H), jnp.float32),
            pltpu.SemaphoreType.DMA,
            pltpu.SemaphoreType.DMA,
        ],
    )
    def body(q0_h, q1_h, s_h, dv_h, par_h, ei_h, zeros_h,
             out_h, t_h, h_h,
             src_v, dst_v, buf0, buf1, q0_v, q1_v, s_v, dv_v, t_v, par_v,
             acc, sem0, sem1):
        cid = lax.axis_index("c")
        sid = lax.axis_index("s")
        wid = cid * NS + sid
        row0 = sid * RPT
        e0 = wid * E_TILE
        rs = pl.ds(row0, RPT)
        pltpu.sync_copy(q0_h.at[rs], q0_v)
        pltpu.sync_copy(q1_h.at[rs], q1_v)
        pltpu.sync_copy(s_h.at[rs], s_v)
        pltpu.sync_copy(dv_h.at[rs], dv_v)
        pltpu.sync_copy(par_h, par_v)
        pltpu.sync_copy(zeros_h.at[rs], acc.at[rs])
        pltpu.sync_copy(ei_h.at[0, pl.ds(e0, E_TILE)], src_v)
        pltpu.sync_copy(ei_h.at[1, pl.ds(e0, E_TILE)], dst_v)

        b0v = par_v[0]
        meanv = par_v[1]
        varv = par_v[2]
        gammav = par_v[3]
        betav = par_v[4]
        scalev = gammav * _nrsqrt(varv + 1e-5)
        shiftv = betav - meanv * scalev

        def row(r, carry):
            dvr = dv_v[r]
            conv = (q0_v[r] + q1_v[r] + s_v[r]) * dvr + b0v
            hrow = conv * scalev + shiftv
            q0_v[r] = hrow
            t_v[r] = hrow * dvr
            return carry

        lax.fori_loop(0, RPT, row, 0)
        pltpu.sync_copy(q0_v, h_h.at[rs])
        pltpu.sync_copy(t_v, t_h.at[rs])
        plsc.subcore_barrier()
        _edge_pass(t_h, src_v, dst_v, buf0, buf1, acc, sem0, sem1)
        plsc.subcore_barrier()
        pltpu.sync_copy(acc.at[rs], out_h.at[cid, rs])

    return body(q0, q1, s, dv, params, ei, zeros)


def _sc_deg(ei, e0c, zeros):
    @functools.partial(
        pl.kernel,
        mesh=_mesh,
        out_type=jax.ShapeDtypeStruct((NC, N_PAD, H), jnp.float32),
        compiler_params=pltpu.CompilerParams(use_tc_tiling_on_sc=False),
        scratch_types=[
            pltpu.VMEM((E_TILE,), jnp.int32),
            pltpu.VMEM((CHUNK, H), jnp.float32),
            pltpu.VMEM_SHARED((N_PAD, H), jnp.float32),
            pltpu.SemaphoreType.DMA,
        ],
    )
    def body(ei_h, e0_h, zeros_h, out_h, dst_v, buf, acc, sem):
        cid = lax.axis_index("c")
        sid = lax.axis_index("s")
        wid = cid * NS + sid
        row0 = sid * RPT
        e0 = wid * E_TILE
        pltpu.sync_copy(zeros_h.at[pl.ds(row0, RPT)], acc.at[pl.ds(row0, RPT)])
        pltpu.sync_copy(ei_h.at[1, pl.ds(e0, E_TILE)], dst_v)
        pltpu.sync_copy(e0_h, buf)
        plsc.subcore_barrier()

        def chunk(j, carry):
            pltpu.sync_copy(buf, acc.at[dst_v.at[pl.ds(j * CHUNK, CHUNK)]],
                            add=True)
            return carry

        lax.fori_loop(0, CHUNKS, chunk, 0)
        tb = CHUNKS * CHUNK
        pltpu.sync_copy(buf.at[pl.ds(0, TAIL)],
                        acc.at[dst_v.at[pl.ds(tb, TAIL)]], add=True)
        plsc.subcore_barrier()
        pltpu.sync_copy(acc.at[pl.ds(row0, RPT)], out_h.at[cid, pl.ds(row0, RPT)])

    return body(ei, e0c, zeros)


def _mm_scale_body(x_ref, w_ref, p0_ref, p1_ref, s_ref, dv_ref):
    h0 = jnp.dot(x_ref[...], w_ref[...], preferred_element_type=jnp.float32)
    deg = p0_ref[:, 0:1] + p1_ref[:, 0:1] + 1.0
    dv = lax.rsqrt(deg)
    s_ref[...] = h0 * dv
    dv_ref[...] = jnp.broadcast_to(dv, (BM, H))


def _tc_mm_scale(x_p, w0, p0, p1):
    blk = pl.BlockSpec((BM, H), lambda i: (i, 0))
    return pl.pallas_call(
        _mm_scale_body,
        grid=(N_PAD // BM,),
        in_specs=[pl.BlockSpec((BM, D_IN), lambda i: (i, 0)),
                  pl.BlockSpec((D_IN, H), lambda i: (0, 0)),
                  blk, blk],
        out_specs=[blk, blk],
        out_shape=[jax.ShapeDtypeStruct((N_PAD, H), jnp.float32),
                   jax.ShapeDtypeStruct((N_PAD, H), jnp.float32)],
    )(x_p, w0, p0, p1)


def _out_body(r0_ref, r1_ref, t_ref, dv_ref, w_ref, b_ref, o_ref):
    u = (r0_ref[...] + r1_ref[...] + t_ref[...]) * dv_ref[...]
    emb = lax.dot_general(w_ref[...], u, (((0,), (1,)), ((), ())),
                          preferred_element_type=jnp.float32) + b_ref[...]
    m = jnp.max(emb, axis=0, keepdims=True)
    z = emb - m
    lse = jnp.log(jnp.sum(jnp.exp(z), axis=0, keepdims=True))
    o_ref[...] = z - lse


def _tc_out(r0, r1, t, dv, w_pad, b_pad):
    blk = pl.BlockSpec((N, H), lambda i: (0, 0))
    return pl.pallas_call(
        _out_body,
        grid=(1,),
        in_specs=[blk, blk, blk, blk,
                  pl.BlockSpec((H, D_OUT_PAD), lambda i: (0, 0)),
                  pl.BlockSpec((D_OUT_PAD, 1), lambda i: (0, 0))],
        out_specs=pl.BlockSpec((D_OUT_PAD, N), lambda i: (0, 0)),
        out_shape=jax.ShapeDtypeStruct((D_OUT_PAD, N), jnp.float32),
        compiler_params=pltpu.CompilerParams(
            vmem_limit_bytes=48 * 1024 * 1024),
    )(r0, r1, t, dv, w_pad, b_pad)


def kernel(x, edge_index, W0, b0, bn_gamma, bn_beta, bn_mean, bn_var,
           W_out, b_out):
    ei = edge_index.astype(jnp.int32)

    x_p = jnp.pad(x, ((0, N_PAD - N), (0, 0)))
    zeros = jnp.zeros((N_PAD, H), jnp.float32)
    e0c = jnp.zeros((CHUNK, H), jnp.float32).at[:, 0].set(1.0)
    w_pad = jnp.pad(W_out, ((0, 0), (0, D_OUT_PAD - D_OUT)))
    b_pad = jnp.concatenate(
        [b_out, jnp.full((D_OUT_PAD - D_OUT,), -1e30, jnp.float32)]
    ).reshape(D_OUT_PAD, 1)
    params = jnp.concatenate(
        [b0, bn_mean, bn_var, bn_gamma, bn_beta,
         jnp.zeros((3 * H,), jnp.float32)]).reshape(8, H)

    degp = _sc_deg(ei, e0c, zeros)
    s, dv = _tc_mm_scale(x_p, W0, degp[0], degp[1])
    aggp = _sc_agg(s, ei, zeros)
    agg2, t, h = _sc_bn_agg(aggp[0], aggp[1], s, dv, params, ei,
                            zeros)
    yt = _tc_out(agg2[0], agg2[1], t, dv, w_pad, b_pad)

    return (h[:N], yt[:D_OUT].T)


# speedup vs baseline: 1.0626x; 1.0626x over previous
import functools

import jax
import jax.numpy as jnp
from jax import lax
from jax.experimental import pallas as pl
from jax.experimental.pallas import tpu as pltpu
from jax.experimental.pallas import tpu_sc as plsc

N = 10000
E = 320000
D_IN = 128
H = 16
D_OUT = 300

NC = 2
NS = 16
NW = NC * NS
E_TILE = E // NW
CHUNK = 128
CHUNKS = E_TILE // CHUNK
TAIL = E_TILE - CHUNKS * CHUNK
N_PAD = 10240
RPT = N_PAD // NS
D_OUT_PAD = 304
BM = 1024
BT = 2000

_mesh = plsc.VectorSubcoreMesh(
    core_axis_name="c", subcore_axis_name="s", num_cores=NC, num_subcores=NS)


def _edge_pass(table_h, src_v, dst_v, buf0, buf1, acc, sem0, sem1):
    def gather(j, buf, sem):
        return pltpu.async_copy(
            table_h.at[src_v.at[pl.ds(j * CHUNK, CHUNK)]], buf, sem)

    def gwait(buf, sem):
        pltpu.make_async_copy(table_h.at[pl.ds(0, CHUNK)], buf, sem).wait()

    def scat(j, buf):
        pltpu.sync_copy(buf, acc.at[dst_v.at[pl.ds(j * CHUNK, CHUNK)]],
                        add=True)

    gather(0, buf0, sem0)

    def pair(i, carry):
        j = 2 * i
        gather(j + 1, buf1, sem1)
        gwait(buf0, sem0)
        scat(j, buf0)

        @pl.when(i < CHUNKS // 2 - 1)
        def _():
            gather(j + 2, buf0, sem0)

        gwait(buf1, sem1)
        scat(j + 1, buf1)
        return carry

    lax.fori_loop(0, CHUNKS // 2, pair, 0)
    tb = CHUNKS * CHUNK
    pltpu.async_copy(
        table_h.at[src_v.at[pl.ds(tb, TAIL)]], buf0.at[pl.ds(0, TAIL)],
        sem0).wait()
    pltpu.sync_copy(buf0.at[pl.ds(0, TAIL)],
                    acc.at[dst_v.at[pl.ds(tb, TAIL)]], add=True)


def _sc_agg(table, ei, zeros):
    @functools.partial(
        pl.kernel,
        mesh=_mesh,
        out_type=jax.ShapeDtypeStruct((NC, N_PAD, H), jnp.float32),
        compiler_params=pltpu.CompilerParams(use_tc_tiling_on_sc=False),
        scratch_types=[
            pltpu.VMEM((E_TILE,), jnp.int32),
            pltpu.VMEM((E_TILE,), jnp.int32),
            pltpu.VMEM((CHUNK, H), jnp.float32),
            pltpu.VMEM((CHUNK, H), jnp.float32),
            pltpu.VMEM_SHARED((N_PAD, H), jnp.float32),
            pltpu.SemaphoreType.DMA,
            pltpu.SemaphoreType.DMA,
        ],
    )
    def body(table_h, ei_h, zeros_h, out_h, src_v, dst_v, buf0, buf1, acc,
             sem0, sem1):
        cid = lax.axis_index("c")
        sid = lax.axis_index("s")
        wid = cid * NS + sid
        row0 = sid * RPT
        e0 = wid * E_TILE
        pltpu.sync_copy(zeros_h.at[pl.ds(row0, RPT)], acc.at[pl.ds(row0, RPT)])
        pltpu.sync_copy(ei_h.at[0, pl.ds(e0, E_TILE)], src_v)
        pltpu.sync_copy(ei_h.at[1, pl.ds(e0, E_TILE)], dst_v)
        plsc.subcore_barrier()
        _edge_pass(table_h, src_v, dst_v, buf0, buf1, acc, sem0, sem1)
        plsc.subcore_barrier()
        pltpu.sync_copy(acc.at[pl.ds(row0, RPT)], out_h.at[cid, pl.ds(row0, RPT)])

    return body(table, ei, zeros)


def _bn_body(q0_ref, q1_ref, s_ref, dv_ref, b0_ref, mean_ref, var_ref,
             gamma_ref, beta_ref, h_ref, t_ref):
    dv = dv_ref[...]
    scale = gamma_ref[...] * lax.rsqrt(var_ref[...] + 1e-5)
    conv = (q0_ref[...] + q1_ref[...] + s_ref[...]) * dv + b0_ref[...]
    h = (conv - mean_ref[...]) * scale + beta_ref[...]
    h_ref[...] = h
    t_ref[...] = h * dv


def _tc_bn(q0, q1, s, dv, b0r, meanr, varr, gammar, betar):
    blk = pl.BlockSpec((BM, H), lambda i: (i, 0))
    par = pl.BlockSpec((1, H), lambda i: (0, 0))
    return pl.pallas_call(
        _bn_body,
        grid=(N_PAD // BM,),
        in_specs=[blk, blk, blk, blk, par, par, par, par, par],
        out_specs=[blk, blk],
        out_shape=[jax.ShapeDtypeStruct((N_PAD, H), jnp.float32),
                   jax.ShapeDtypeStruct((N_PAD, H), jnp.float32)],
    )(q0, q1, s, dv, b0r, meanr, varr, gammar, betar)


def _sc_deg(ei, e0c, zeros):
    @functools.partial(
        pl.kernel,
        mesh=_mesh,
        out_type=jax.ShapeDtypeStruct((NC, N_PAD, H), jnp.float32),
        compiler_params=pltpu.CompilerParams(use_tc_tiling_on_sc=False),
        scratch_types=[
            pltpu.VMEM((E_TILE,), jnp.int32),
            pltpu.VMEM((CHUNK, H), jnp.float32),
            pltpu.VMEM_SHARED((N_PAD, H), jnp.float32),
            pltpu.SemaphoreType.DMA,
        ],
    )
    def body(ei_h, e0_h, zeros_h, out_h, dst_v, buf, acc, sem):
        cid = lax.axis_index("c")
        sid = lax.axis_index("s")
        wid = cid * NS + sid
        row0 = sid * RPT
        e0 = wid * E_TILE
        pltpu.sync_copy(zeros_h.at[pl.ds(row0, RPT)], acc.at[pl.ds(row0, RPT)])
        pltpu.sync_copy(ei_h.at[1, pl.ds(e0, E_TILE)], dst_v)
        pltpu.sync_copy(e0_h, buf)
        plsc.subcore_barrier()

        def chunk(j, carry):
            pltpu.async_copy(buf, acc.at[dst_v.at[pl.ds(j * CHUNK, CHUNK)]],
                             sem, add=True)
            return carry

        lax.fori_loop(0, CHUNKS, chunk, 0)
        tb = CHUNKS * CHUNK
        pltpu.async_copy(buf.at[pl.ds(0, TAIL)],
                         acc.at[dst_v.at[pl.ds(tb, TAIL)]], sem, add=True)

        def drain(j, carry):
            pltpu.make_async_copy(
                buf, acc.at[dst_v.at[pl.ds(0, CHUNK)]], sem).wait()
            return carry

        lax.fori_loop(0, CHUNKS, drain, 0)
        pltpu.make_async_copy(
            buf.at[pl.ds(0, TAIL)], acc.at[dst_v.at[pl.ds(0, TAIL)]],
            sem).wait()
        plsc.subcore_barrier()
        pltpu.sync_copy(acc.at[pl.ds(row0, RPT)], out_h.at[cid, pl.ds(row0, RPT)])

    return body(ei, e0c, zeros)


def _mm_body(x_ref, w_ref, o_ref):
    o_ref[...] = jnp.dot(x_ref[...], w_ref[...],
                         preferred_element_type=jnp.float32)


def _tc_matmul(x_p, w0):
    return pl.pallas_call(
        _mm_body,
        grid=(N_PAD // BM,),
        in_specs=[pl.BlockSpec((BM, D_IN), lambda i: (i, 0)),
                  pl.BlockSpec((D_IN, H), lambda i: (0, 0))],
        out_specs=pl.BlockSpec((BM, H), lambda i: (i, 0)),
        out_shape=jax.ShapeDtypeStruct((N_PAD, H), jnp.float32),
    )(x_p, w0)


def _scale_body(p0_ref, p1_ref, h0_ref, s_ref, dv_ref):
    deg = p0_ref[:, 0:1] + p1_ref[:, 0:1] + 1.0
    dv = lax.rsqrt(deg)
    s_ref[...] = h0_ref[...] * dv
    dv_ref[...] = jnp.broadcast_to(dv, (BM, H))


def _tc_scale(p0, p1, h0):
    blk = pl.BlockSpec((BM, H), lambda i: (i, 0))
    return pl.pallas_call(
        _scale_body,
        grid=(N_PAD // BM,),
        in_specs=[blk, blk, blk],
        out_specs=[blk, blk],
        out_shape=[jax.ShapeDtypeStruct((N_PAD, H), jnp.float32),
                   jax.ShapeDtypeStruct((N_PAD, H), jnp.float32)],
    )(p0, p1, h0)


def _out_body(r0_ref, r1_ref, t_ref, dv_ref, w_ref, b_ref, o_ref):
    u = (r0_ref[...] + r1_ref[...] + t_ref[...]) * dv_ref[...]
    emb = lax.dot_general(w_ref[...], u, (((0,), (1,)), ((), ())),
                          preferred_element_type=jnp.float32) + b_ref[...]
    m = jnp.max(emb, axis=0, keepdims=True)
    z = emb - m
    lse = jnp.log(jnp.sum(jnp.exp(z), axis=0, keepdims=True))
    o_ref[...] = z - lse


def _tc_out(r0, r1, t, dv, w_pad, b_pad):
    blk = pl.BlockSpec((N, H), lambda i: (0, 0))
    return pl.pallas_call(
        _out_body,
        grid=(1,),
        in_specs=[blk, blk, blk, blk,
                  pl.BlockSpec((H, D_OUT_PAD), lambda i: (0, 0)),
                  pl.BlockSpec((D_OUT_PAD, 1), lambda i: (0, 0))],
        out_specs=pl.BlockSpec((D_OUT_PAD, N), lambda i: (0, 0)),
        out_shape=jax.ShapeDtypeStruct((D_OUT_PAD, N), jnp.float32),
        compiler_params=pltpu.CompilerParams(
            vmem_limit_bytes=48 * 1024 * 1024),
    )(r0, r1, t, dv, w_pad, b_pad)


def kernel(x, edge_index, W0, b0, bn_gamma, bn_beta, bn_mean, bn_var,
           W_out, b_out):
    ei = edge_index.astype(jnp.int32)

    x_p = jnp.pad(x, ((0, N_PAD - N), (0, 0)))
    zeros = jnp.zeros((N_PAD, H), jnp.float32)
    e0c = jnp.zeros((CHUNK, H), jnp.float32).at[:, 0].set(1.0)
    w_pad = jnp.pad(W_out, ((0, 0), (0, D_OUT_PAD - D_OUT)))
    b_pad = jnp.concatenate(
        [b_out, jnp.full((D_OUT_PAD - D_OUT,), -1e30, jnp.float32)]
    ).reshape(D_OUT_PAD, 1)
    b0r = b0.reshape(1, H)
    meanr = bn_mean.reshape(1, H)
    varr = bn_var.reshape(1, H)
    gammar = bn_gamma.reshape(1, H)
    betar = bn_beta.reshape(1, H)

    h0 = _tc_matmul(x_p, W0)
    degp = _sc_deg(ei, e0c, zeros)
    s, dv = _tc_scale(degp[0], degp[1], h0)
    aggp = _sc_agg(s, ei, zeros)
    h, t = _tc_bn(aggp[0], aggp[1], s, dv,
                  b0r, meanr, varr, gammar, betar)
    agg2 = _sc_agg(t, ei, zeros)
    yt = _tc_out(agg2[0], agg2[1], t, dv, w_pad, b_pad)

    return (h[:N], yt[:D_OUT].T)


# speedup vs baseline: 1.2374x; 1.1645x over previous
import functools

import jax
import jax.numpy as jnp
from jax import lax
from jax.experimental import pallas as pl
from jax.experimental.pallas import tpu as pltpu
from jax.experimental.pallas import tpu_sc as plsc

N = 10000
E = 320000
D_IN = 128
H = 16
D_OUT = 300

NC = 2
NS = 16
NW = NC * NS
E_TILE = E // NW
CHUNK = 128
CHUNKS = E_TILE // CHUNK
TAIL = E_TILE - CHUNKS * CHUNK
N_PAD = 10240
RPT = N_PAD // NS
D_OUT_PAD = 304
BM = 1024
BT = 2000

_mesh = plsc.VectorSubcoreMesh(
    core_axis_name="c", subcore_axis_name="s", num_cores=NC, num_subcores=NS)


def _edge_pass(table_h, src_v, dst_v, bufs, acc, sems):
    def gather(j, buf, sem):
        return pltpu.async_copy(
            table_h.at[src_v.at[pl.ds(j * CHUNK, CHUNK)]], buf, sem)

    def gwait(buf, sem):
        pltpu.make_async_copy(table_h.at[pl.ds(0, CHUNK)], buf, sem).wait()

    def scat(j, buf):
        pltpu.sync_copy(buf, acc.at[dst_v.at[pl.ds(j * CHUNK, CHUNK)]],
                        add=True)

    nb = len(bufs)
    for k in range(nb):
        gather(k, bufs[k], sems[k])

    def quad(i, carry):
        j = nb * i
        for k in range(nb):
            gwait(bufs[k], sems[k])
            scat(j + k, bufs[k])

            @pl.when(j + k + nb < CHUNKS)
            def _():
                gather(j + k + nb, bufs[k], sems[k])

        return carry

    lax.fori_loop(0, CHUNKS // nb, quad, 0)
    for k in range(CHUNKS - nb * (CHUNKS // nb)):
        jj = nb * (CHUNKS // nb) + k
        gwait(bufs[k], sems[k])
        scat(jj, bufs[k])
    tb = CHUNKS * CHUNK
    pltpu.async_copy(
        table_h.at[src_v.at[pl.ds(tb, TAIL)]], bufs[0].at[pl.ds(0, TAIL)],
        sems[0]).wait()
    pltpu.sync_copy(bufs[0].at[pl.ds(0, TAIL)],
                    acc.at[dst_v.at[pl.ds(tb, TAIL)]], add=True)


def _sc_agg(table, ei, zeros):
    @functools.partial(
        pl.kernel,
        mesh=_mesh,
        out_type=jax.ShapeDtypeStruct((NC, N_PAD, H), jnp.float32),
        compiler_params=pltpu.CompilerParams(use_tc_tiling_on_sc=False),
        scratch_types=[
            pltpu.VMEM((E_TILE,), jnp.int32),
            pltpu.VMEM((E_TILE,), jnp.int32),
            pltpu.VMEM((CHUNK, H), jnp.float32),
            pltpu.VMEM((CHUNK, H), jnp.float32),
            pltpu.VMEM((CHUNK, H), jnp.float32),
            pltpu.VMEM((CHUNK, H), jnp.float32),
            pltpu.VMEM_SHARED((N_PAD, H), jnp.float32),
            pltpu.SemaphoreType.DMA,
            pltpu.SemaphoreType.DMA,
            pltpu.SemaphoreType.DMA,
            pltpu.SemaphoreType.DMA,
        ],
    )
    def body(table_h, ei_h, zeros_h, out_h, src_v, dst_v, buf0, buf1, buf2,
             buf3, acc, sem0, sem1, sem2, sem3):
        cid = lax.axis_index("c")
        sid = lax.axis_index("s")
        wid = cid * NS + sid
        row0 = sid * RPT
        e0 = wid * E_TILE
        pltpu.sync_copy(zeros_h.at[pl.ds(row0, RPT)], acc.at[pl.ds(row0, RPT)])
        pltpu.sync_copy(ei_h.at[0, pl.ds(e0, E_TILE)], src_v)
        pltpu.sync_copy(ei_h.at[1, pl.ds(e0, E_TILE)], dst_v)
        plsc.subcore_barrier()
        _edge_pass(table_h, src_v, dst_v, (buf0, buf1, buf2, buf3), acc,
                   (sem0, sem1, sem2, sem3))
        plsc.subcore_barrier()
        pltpu.sync_copy(acc.at[pl.ds(row0, RPT)], out_h.at[cid, pl.ds(row0, RPT)])

    return body(table, ei, zeros)


def _bn_body(q0_ref, q1_ref, s_ref, dv_ref, b0_ref, mean_ref, var_ref,
             gamma_ref, beta_ref, h_ref, t_ref):
    dv = dv_ref[...]
    scale = gamma_ref[...] * lax.rsqrt(var_ref[...] + 1e-5)
    conv = (q0_ref[...] + q1_ref[...] + s_ref[...]) * dv + b0_ref[...]
    h = (conv - mean_ref[...]) * scale + beta_ref[...]
    h_ref[...] = h
    t_ref[...] = h * dv


def _tc_bn(q0, q1, s, dv, b0r, meanr, varr, gammar, betar):
    blk = pl.BlockSpec((BM, H), lambda i: (i, 0))
    par = pl.BlockSpec((1, H), lambda i: (0, 0))
    return pl.pallas_call(
        _bn_body,
        grid=(N_PAD // BM,),
        in_specs=[blk, blk, blk, blk, par, par, par, par, par],
        out_specs=[blk, blk],
        out_shape=[jax.ShapeDtypeStruct((N_PAD, H), jnp.float32),
                   jax.ShapeDtypeStruct((N_PAD, H), jnp.float32)],
    )(q0, q1, s, dv, b0r, meanr, varr, gammar, betar)


def _sc_deg(ei, e0c, zeros):
    @functools.partial(
        pl.kernel,
        mesh=_mesh,
        out_type=jax.ShapeDtypeStruct((NC, N_PAD, H), jnp.float32),
        compiler_params=pltpu.CompilerParams(use_tc_tiling_on_sc=False),
        scratch_types=[
            pltpu.VMEM((E_TILE,), jnp.int32),
            pltpu.VMEM((CHUNK, H), jnp.float32),
            pltpu.VMEM_SHARED((N_PAD, H), jnp.float32),
            pltpu.SemaphoreType.DMA,
        ],
    )
    def body(ei_h, e0_h, zeros_h, out_h, dst_v, buf, acc, sem):
        cid = lax.axis_index("c")
        sid = lax.axis_index("s")
        wid = cid * NS + sid
        row0 = sid * RPT
        e0 = wid * E_TILE
        pltpu.sync_copy(zeros_h.at[pl.ds(row0, RPT)], acc.at[pl.ds(row0, RPT)])
        pltpu.sync_copy(ei_h.at[1, pl.ds(e0, E_TILE)], dst_v)
        pltpu.sync_copy(e0_h, buf)
        plsc.subcore_barrier()

        def chunk(j, carry):
            pltpu.async_copy(buf, acc.at[dst_v.at[pl.ds(j * CHUNK, CHUNK)]],
                             sem, add=True)
            return carry

        lax.fori_loop(0, CHUNKS, chunk, 0)
        tb = CHUNKS * CHUNK
        pltpu.async_copy(buf.at[pl.ds(0, TAIL)],
                         acc.at[dst_v.at[pl.ds(tb, TAIL)]], sem, add=True)

        def drain(j, carry):
            pltpu.make_async_copy(
                buf, acc.at[dst_v.at[pl.ds(0, CHUNK)]], sem).wait()
            return carry

        lax.fori_loop(0, CHUNKS, drain, 0)
        pltpu.make_async_copy(
            buf.at[pl.ds(0, TAIL)], acc.at[dst_v.at[pl.ds(0, TAIL)]],
            sem).wait()
        plsc.subcore_barrier()
        pltpu.sync_copy(acc.at[pl.ds(row0, RPT)], out_h.at[cid, pl.ds(row0, RPT)])

    return body(ei, e0c, zeros)


def _mm_body(x_ref, w_ref, o_ref):
    o_ref[...] = jnp.dot(x_ref[...], w_ref[...],
                         preferred_element_type=jnp.float32)


def _tc_matmul(x_p, w0):
    return pl.pallas_call(
        _mm_body,
        grid=(N_PAD // BM,),
        in_specs=[pl.BlockSpec((BM, D_IN), lambda i: (i, 0)),
                  pl.BlockSpec((D_IN, H), lambda i: (0, 0))],
        out_specs=pl.BlockSpec((BM, H), lambda i: (i, 0)),
        out_shape=jax.ShapeDtypeStruct((N_PAD, H), jnp.float32),
    )(x_p, w0)


def _scale_body(p0_ref, p1_ref, h0_ref, s_ref, dv_ref):
    deg = p0_ref[:, 0:1] + p1_ref[:, 0:1] + 1.0
    dv = lax.rsqrt(deg)
    s_ref[...] = h0_ref[...] * dv
    dv_ref[...] = jnp.broadcast_to(dv, (BM, H))


def _tc_scale(p0, p1, h0):
    blk = pl.BlockSpec((BM, H), lambda i: (i, 0))
    return pl.pallas_call(
        _scale_body,
        grid=(N_PAD // BM,),
        in_specs=[blk, blk, blk],
        out_specs=[blk, blk],
        out_shape=[jax.ShapeDtypeStruct((N_PAD, H), jnp.float32),
                   jax.ShapeDtypeStruct((N_PAD, H), jnp.float32)],
    )(p0, p1, h0)


def _out_body(r0_ref, r1_ref, t_ref, dv_ref, w_ref, b_ref, o_ref):
    u = (r0_ref[...] + r1_ref[...] + t_ref[...]) * dv_ref[...]
    emb = lax.dot_general(w_ref[...], u, (((0,), (1,)), ((), ())),
                          preferred_element_type=jnp.float32) + b_ref[...]
    m = jnp.max(emb, axis=0, keepdims=True)
    z = emb - m
    lse = jnp.log(jnp.sum(jnp.exp(z), axis=0, keepdims=True))
    o_ref[...] = z - lse


def _tc_out(r0, r1, t, dv, w_pad, b_pad):
    blk = pl.BlockSpec((N, H), lambda i: (0, 0))
    return pl.pallas_call(
        _out_body,
        grid=(1,),
        in_specs=[blk, blk, blk, blk,
                  pl.BlockSpec((H, D_OUT_PAD), lambda i: (0, 0)),
                  pl.BlockSpec((D_OUT_PAD, 1), lambda i: (0, 0))],
        out_specs=pl.BlockSpec((D_OUT_PAD, N), lambda i: (0, 0)),
        out_shape=jax.ShapeDtypeStruct((D_OUT_PAD, N), jnp.float32),
        compiler_params=pltpu.CompilerParams(
            vmem_limit_bytes=48 * 1024 * 1024),
    )(r0, r1, t, dv, w_pad, b_pad)


def kernel(x, edge_index, W0, b0, bn_gamma, bn_beta, bn_mean, bn_var,
           W_out, b_out):
    ei = edge_index.astype(jnp.int32)

    x_p = jnp.pad(x, ((0, N_PAD - N), (0, 0)))
    zeros = jnp.zeros((N_PAD, H), jnp.float32)
    e0c = jnp.zeros((CHUNK, H), jnp.float32).at[:, 0].set(1.0)
    w_pad = jnp.pad(W_out, ((0, 0), (0, D_OUT_PAD - D_OUT)))
    b_pad = jnp.concatenate(
        [b_out, jnp.full((D_OUT_PAD - D_OUT,), -1e30, jnp.float32)]
    ).reshape(D_OUT_PAD, 1)
    b0r = b0.reshape(1, H)
    meanr = bn_mean.reshape(1, H)
    varr = bn_var.reshape(1, H)
    gammar = bn_gamma.reshape(1, H)
    betar = bn_beta.reshape(1, H)

    h0 = _tc_matmul(x_p, W0)
    degp = _sc_deg(ei, e0c, zeros)
    s, dv = _tc_scale(degp[0], degp[1], h0)
    aggp = _sc_agg(s, ei, zeros)
    h, t = _tc_bn(aggp[0], aggp[1], s, dv,
                  b0r, meanr, varr, gammar, betar)
    agg2 = _sc_agg(t, ei, zeros)
    yt = _tc_out(agg2[0], agg2[1], t, dv, w_pad, b_pad)

    return (h[:N], yt[:D_OUT].T)
